# baseline (speedup 1.0000x reference)
import math

import jax
import jax.numpy as jnp
from jax.experimental import pallas as pl
from jax.experimental.pallas import tpu as pltpu

B, S, D = 1, 2048, 768
H, E, TOPK, P = 12, 8, 2, 64
TS = 256
NT = S // TS
HP = H * P

_SCALE = 1.0 / math.sqrt(P)
_S = math.sqrt(_SCALE)

_HI = jax.lax.Precision.HIGHEST


def _prep_body(v_ref, o_ref, vmat_ref, omat_ref):
    for h in range(H):
        vmat_ref[0, :, h * P:(h + 1) * P] = v_ref[h, 0].astype(jnp.bfloat16)
        omat_ref[0, h * P:(h + 1) * P, :] = o_ref[h, 0].astype(jnp.bfloat16)


def _top2_gates(logits, rs_over):
    probs = [jax.nn.sigmoid(logits[:, e * H:(e + 1) * H]) for e in range(E)]
    m1 = probs[0]
    for e in range(1, E):
        m1 = jnp.maximum(m1, probs[e])
    i1 = jnp.full(probs[0].shape, E, dtype=jnp.int32)
    for e in range(E - 1, -1, -1):
        i1 = jnp.where(probs[e] == m1, e, i1)
    neg = jnp.float32(-jnp.inf)
    q = [jnp.where(i1 == e, neg, probs[e]) for e in range(E)]
    m2 = q[0]
    for e in range(1, E):
        m2 = jnp.maximum(m2, q[e])
    i2 = jnp.full(probs[0].shape, E, dtype=jnp.int32)
    for e in range(E - 1, -1, -1):
        i2 = jnp.where(q[e] == m2, e, i2)
    denom = jnp.maximum(m1 + m2, jnp.float32(1e-9))
    scale = rs_over / denom
    gates = []
    for e in range(E):
        sel = jnp.logical_or(i1 == e, i2 == e)
        gates.append(jnp.where(sel, probs[e] * scale, jnp.float32(0.0)))
    return gates


def _proj_route_body(rs_ref, x_ref, wq_ref, wk_ref, svt_ref, sot_ref,
                     vmat_ref, e12_ref,
                     q_ref, k_ref, vmix_ref, go_ref):
    xb = x_ref[...]
    s = jnp.float32(_S)
    dn = (((1,), (0,)), ((), ()))
    dnt = (((1,), (1,)), ((), ()))
    q = jax.lax.dot_general(xb, wq_ref[...], dnt,
                            preferred_element_type=jnp.float32)
    q_ref[...] = (q * s).astype(jnp.bfloat16)
    k = jax.lax.dot_general(xb, wk_ref[...], dnt,
                            preferred_element_type=jnp.float32)
    k_ref[...] = (k * s).astype(jnp.bfloat16)

    rs = rs_ref[0, 0]
    lv = jax.lax.dot_general(xb, svt_ref[...], dnt,
                             preferred_element_type=jnp.float32)
    gv = _top2_gates(lv, rs)
    lo = jax.lax.dot_general(xb, sot_ref[...], dnt,
                             preferred_element_type=jnp.float32)
    go = _top2_gates(lo, rs)
    for e in range(E):
        go_ref[:, e * H:(e + 1) * H] = go[e]

    e12 = e12_ref[...]
    acc = jnp.zeros((TS, HP), jnp.float32)
    for e in range(E):
        av = jax.lax.dot_general(xb, vmat_ref[e], dn,
                                 preferred_element_type=jnp.float32)
        gexp = _expand_gate(gv[e], e12)
        acc = acc + av * gexp
    vmix_ref[...] = acc.astype(jnp.bfloat16)


def _expand_gate(g, e12):
    return jax.lax.dot_general(g.astype(jnp.bfloat16), e12,
                               (((1,), (0,)), ((), ())),
                               preferred_element_type=jnp.float32)


def _make_attn_body(q_start, klen):
    def body(q_ref, k_ref, v_ref, o_ref):
        i = pl.program_id(1)
        row = (q_start + i) * TS + jax.lax.broadcasted_iota(
            jnp.int32, (TS, klen), 0)
        col = jax.lax.broadcasted_iota(jnp.int32, (TS, klen), 1)
        mask = col <= row
        for j in range(H):
            qv = q_ref[:, j * P:(j + 1) * P]
            kv = k_ref[:, j * P:(j + 1) * P]
            s = jax.lax.dot_general(qv, kv, (((1,), (1,)), ((), ())),
                                    preferred_element_type=jnp.float32)
            s = jnp.where(mask, s, jnp.float32(-1e30))
            m = jnp.max(s, axis=1, keepdims=True)
            p = jnp.exp(s - m)
            l = jnp.sum(p, axis=1, keepdims=True)
            p = (p / l).astype(jnp.bfloat16)
            o = jax.lax.dot_general(p, v_ref[:, j * P:(j + 1) * P],
                                    (((1,), (0,)), ((), ())),
                                    preferred_element_type=jnp.float32)
            o_ref[:, j * P:(j + 1) * P] = o.astype(jnp.bfloat16)
    return body


def _oproj_body(res_ref, go_ref, omat_ref, e12_ref, out_ref):
    res = res_ref[...].astype(jnp.float32)
    e12 = e12_ref[...]
    dn = (((1,), (0,)), ((), ()))
    acc = jnp.zeros((TS, D), jnp.float32)
    for e in range(E):
        gexp = _expand_gate(go_ref[:, e * H:(e + 1) * H], e12)
        wres = (res * gexp).astype(jnp.bfloat16)
        acc = acc + jax.lax.dot_general(wres, omat_ref[e], dn,
                                        preferred_element_type=jnp.float32)
    out_ref[...] = acc


@jax.jit
def kernel(x, Wq, Wk, v, o, sel_v, sel_o, route_scale):
    xbf = x[0].astype(jnp.bfloat16)
    wqb = Wq.astype(jnp.bfloat16)
    wkb = Wk.astype(jnp.bfloat16)
    svt = sel_v.reshape(H, E, D).transpose(1, 0, 2).reshape(E * H, D)
    svt = svt.astype(jnp.bfloat16)
    sot = sel_o.reshape(H, E, D).transpose(1, 0, 2).reshape(E * H, D)
    sot = sot.astype(jnp.bfloat16)
    rs = route_scale.reshape(1, 1)
    e12 = jnp.repeat(jnp.eye(H, dtype=jnp.bfloat16), P, axis=1)

    def full(shape):
        return pl.BlockSpec(shape, lambda *_: (0,) * len(shape))

    vmat, omat = pl.pallas_call(
        _prep_body,
        grid=(E,),
        in_specs=[
            pl.BlockSpec((H, 1, D, P), lambda e: (0, e, 0, 0)),
            pl.BlockSpec((H, 1, P, D), lambda e: (0, e, 0, 0)),
        ],
        out_specs=[
            pl.BlockSpec((1, D, HP), lambda e: (e, 0, 0)),
            pl.BlockSpec((1, HP, D), lambda e: (e, 0, 0)),
        ],
        out_shape=[
            jax.ShapeDtypeStruct((E, D, HP), jnp.bfloat16),
            jax.ShapeDtypeStruct((E, HP, D), jnp.bfloat16),
        ],
        compiler_params=pltpu.CompilerParams(
            dimension_semantics=("parallel",)),
    )(v.reshape(H, E, D, P), o.reshape(H, E, P, D))

    qk, kk, vmixk, gok = pl.pallas_call(
        _proj_route_body,
        grid=(NT,),
        in_specs=[
            pl.BlockSpec(memory_space=pltpu.SMEM),
            pl.BlockSpec((TS, D), lambda i: (i, 0)),
            full((HP, D)),
            full((HP, D)),
            full((E * H, D)),
            full((E * H, D)),
            full((E, D, HP)),
            full((H, HP)),
        ],
        out_specs=[
            pl.BlockSpec((TS, HP), lambda i: (i, 0)),
            pl.BlockSpec((TS, HP), lambda i: (i, 0)),
            pl.BlockSpec((TS, HP), lambda i: (i, 0)),
            pl.BlockSpec((TS, E * H), lambda i: (i, 0)),
        ],
        out_shape=[
            jax.ShapeDtypeStruct((S, HP), jnp.bfloat16),
            jax.ShapeDtypeStruct((S, HP), jnp.bfloat16),
            jax.ShapeDtypeStruct((S, HP), jnp.bfloat16),
            jax.ShapeDtypeStruct((S, E * H), jnp.float32),
        ],
        compiler_params=pltpu.CompilerParams(
            dimension_semantics=("parallel",)),
    )(rs, xbf, wqb, wkb, svt, sot, vmat, e12)

    parts = []
    for ci in range(4):
        q_start, klen = 2 * ci, (2 * ci + 2) * TS
        parts.append(pl.pallas_call(
            _make_attn_body(q_start, klen),
            grid=(1, 2),
            in_specs=[
                pl.BlockSpec((TS, HP),
                             lambda h, i, qs=q_start: (qs + i, 0)),
                pl.BlockSpec((klen, HP), lambda h, i: (0, 0)),
                pl.BlockSpec((klen, HP), lambda h, i: (0, 0)),
            ],
            out_specs=pl.BlockSpec((TS, HP), lambda h, i: (i, 0)),
            out_shape=jax.ShapeDtypeStruct((2 * TS, HP), jnp.bfloat16),
            compiler_params=pltpu.CompilerParams(
                dimension_semantics=("parallel", "parallel")),
        )(qk, kk, vmixk))
    res = jnp.concatenate(parts, axis=0)

    out = pl.pallas_call(
        _oproj_body,
        grid=(NT,),
        in_specs=[
            pl.BlockSpec((TS, HP), lambda i: (i, 0)),
            pl.BlockSpec((TS, E * H), lambda i: (i, 0)),
            full((E, HP, D)),
            full((H, HP)),
        ],
        out_specs=pl.BlockSpec((TS, D), lambda i: (i, 0)),
        out_shape=jax.ShapeDtypeStruct((S, D), jnp.float32),
        compiler_params=pltpu.CompilerParams(
            dimension_semantics=("parallel",)),
    )(res, gok, omat, e12)

    return out.reshape(B, S, D)


# speedup vs baseline: 1.0115x; 1.0115x over previous
import math

import jax
import jax.numpy as jnp
from jax.experimental import pallas as pl
from jax.experimental.pallas import tpu as pltpu

B, S, D = 1, 2048, 768
H, E, TOPK, P = 12, 8, 2, 64
TS = 256
NT = S // TS
HP = H * P

_SCALE = 1.0 / math.sqrt(P)
_S = math.sqrt(_SCALE)

_HI = jax.lax.Precision.HIGHEST


def _prep_body(v_ref, o_ref, x_ref, vmat_ref, omat_ref, xb_ref):
    for h in range(H):
        vmat_ref[0, :, h * P:(h + 1) * P] = v_ref[h, 0].astype(jnp.bfloat16)
        omat_ref[0, h * P:(h + 1) * P, :] = o_ref[h, 0].astype(jnp.bfloat16)
    xb_ref[...] = x_ref[...].astype(jnp.bfloat16)


def _top2_gates(logits, rs_over):
    probs = [jax.nn.sigmoid(logits[:, e * H:(e + 1) * H]) for e in range(E)]
    m1 = probs[0]
    for e in range(1, E):
        m1 = jnp.maximum(m1, probs[e])
    i1 = jnp.full(probs[0].shape, E, dtype=jnp.int32)
    for e in range(E - 1, -1, -1):
        i1 = jnp.where(probs[e] == m1, e, i1)
    neg = jnp.float32(-jnp.inf)
    q = [jnp.where(i1 == e, neg, probs[e]) for e in range(E)]
    m2 = q[0]
    for e in range(1, E):
        m2 = jnp.maximum(m2, q[e])
    i2 = jnp.full(probs[0].shape, E, dtype=jnp.int32)
    for e in range(E - 1, -1, -1):
        i2 = jnp.where(q[e] == m2, e, i2)
    denom = jnp.maximum(m1 + m2, jnp.float32(1e-9))
    scale = rs_over / denom
    gates = []
    for e in range(E):
        sel = jnp.logical_or(i1 == e, i2 == e)
        gates.append(jnp.where(sel, probs[e] * scale, jnp.float32(0.0)))
    return gates


def _proj_route_body(rs_ref, x_ref, wq_ref, wk_ref, svt_ref, sot_ref,
                     vmat_ref, e12_ref,
                     q_ref, k_ref, vmix_ref, go_ref):
    xb = x_ref[...]
    s = jnp.float32(_S)
    dn = (((1,), (0,)), ((), ()))
    dnt = (((1,), (1,)), ((), ()))
    q = jax.lax.dot_general(xb, wq_ref[...], dnt,
                            preferred_element_type=jnp.float32)
    q_ref[...] = (q * s).astype(jnp.bfloat16)
    k = jax.lax.dot_general(xb, wk_ref[...], dnt,
                            preferred_element_type=jnp.float32)
    k_ref[...] = (k * s).astype(jnp.bfloat16)

    rs = rs_ref[0, 0]
    lv = jax.lax.dot_general(xb, svt_ref[...], dnt,
                             preferred_element_type=jnp.float32)
    gv = _top2_gates(lv, rs)
    lo = jax.lax.dot_general(xb, sot_ref[...], dnt,
                             preferred_element_type=jnp.float32)
    go = _top2_gates(lo, rs)
    for e in range(E):
        go_ref[:, e * H:(e + 1) * H] = go[e]

    e12 = e12_ref[...]
    acc = jnp.zeros((TS, HP), jnp.float32)
    for e in range(E):
        av = jax.lax.dot_general(xb, vmat_ref[e], dn,
                                 preferred_element_type=jnp.float32)
        gexp = _expand_gate(gv[e], e12)
        acc = acc + av * gexp
    vmix_ref[...] = acc.astype(jnp.bfloat16)


def _expand_gate(g, e12):
    return jax.lax.dot_general(g.astype(jnp.bfloat16), e12,
                               (((1,), (0,)), ((), ())),
                               preferred_element_type=jnp.float32)


def _make_attn_body(q_start, klen):
    def body(q_ref, k_ref, v_ref, o_ref):
        i = pl.program_id(1)
        row = (q_start + i) * TS + jax.lax.broadcasted_iota(
            jnp.int32, (TS, klen), 0)
        col = jax.lax.broadcasted_iota(jnp.int32, (TS, klen), 1)
        mask = col <= row
        for j in range(4):
            qv = q_ref[:, j * P:(j + 1) * P]
            kv = k_ref[:, j * P:(j + 1) * P]
            s = jax.lax.dot_general(qv, kv, (((1,), (1,)), ((), ())),
                                    preferred_element_type=jnp.float32)
            s = jnp.where(mask, s, jnp.float32(-1e30))
            m = jnp.max(s, axis=1, keepdims=True)
            p = jnp.exp(s - m)
            l = jnp.sum(p, axis=1, keepdims=True)
            p = (p / l).astype(jnp.bfloat16)
            o = jax.lax.dot_general(p, v_ref[:, j * P:(j + 1) * P],
                                    (((1,), (0,)), ((), ())),
                                    preferred_element_type=jnp.float32)
            o_ref[:, j * P:(j + 1) * P] = o.astype(jnp.bfloat16)
    return body


def _oproj_body(res_ref, go_ref, omat_ref, e12_ref, out_ref):
    res = res_ref[...].astype(jnp.float32)
    e12 = e12_ref[...]
    dn = (((1,), (0,)), ((), ()))
    acc = jnp.zeros((TS, D), jnp.float32)
    for e in range(E):
        gexp = _expand_gate(go_ref[:, e * H:(e + 1) * H], e12)
        wres = (res * gexp).astype(jnp.bfloat16)
        acc = acc + jax.lax.dot_general(wres, omat_ref[e], dn,
                                        preferred_element_type=jnp.float32)
    out_ref[...] = acc


@jax.jit
def kernel(x, Wq, Wk, v, o, sel_v, sel_o, route_scale):
    wqb = Wq.astype(jnp.bfloat16)
    wkb = Wk.astype(jnp.bfloat16)
    svt = sel_v.reshape(H, E, D).transpose(1, 0, 2).reshape(E * H, D)
    svt = svt.astype(jnp.bfloat16)
    sot = sel_o.reshape(H, E, D).transpose(1, 0, 2).reshape(E * H, D)
    sot = sot.astype(jnp.bfloat16)
    rs = route_scale.reshape(1, 1)
    e12 = jnp.repeat(jnp.eye(H, dtype=jnp.bfloat16), P, axis=1)

    def full(shape):
        return pl.BlockSpec(shape, lambda *_: (0,) * len(shape))

    vmat, omat, xbf = pl.pallas_call(
        _prep_body,
        grid=(E,),
        in_specs=[
            pl.BlockSpec((H, 1, D, P), lambda e: (0, e, 0, 0)),
            pl.BlockSpec((H, 1, P, D), lambda e: (0, e, 0, 0)),
            pl.BlockSpec((TS, D), lambda e: (e, 0)),
        ],
        out_specs=[
            pl.BlockSpec((1, D, HP), lambda e: (e, 0, 0)),
            pl.BlockSpec((1, HP, D), lambda e: (e, 0, 0)),
            pl.BlockSpec((TS, D), lambda e: (e, 0)),
        ],
        out_shape=[
            jax.ShapeDtypeStruct((E, D, HP), jnp.bfloat16),
            jax.ShapeDtypeStruct((E, HP, D), jnp.bfloat16),
            jax.ShapeDtypeStruct((S, D), jnp.bfloat16),
        ],
        compiler_params=pltpu.CompilerParams(
            dimension_semantics=("parallel",)),
    )(v.reshape(H, E, D, P), o.reshape(H, E, P, D), x[0])

    qk, kk, vmixk, gok = pl.pallas_call(
        _proj_route_body,
        grid=(NT,),
        in_specs=[
            pl.BlockSpec(memory_space=pltpu.SMEM),
            pl.BlockSpec((TS, D), lambda i: (i, 0)),
            full((HP, D)),
            full((HP, D)),
            full((E * H, D)),
            full((E * H, D)),
            full((E, D, HP)),
            full((H, HP)),
        ],
        out_specs=[
            pl.BlockSpec((TS, HP), lambda i: (i, 0)),
            pl.BlockSpec((TS, HP), lambda i: (i, 0)),
            pl.BlockSpec((TS, HP), lambda i: (i, 0)),
            pl.BlockSpec((TS, E * H), lambda i: (i, 0)),
        ],
        out_shape=[
            jax.ShapeDtypeStruct((S, HP), jnp.bfloat16),
            jax.ShapeDtypeStruct((S, HP), jnp.bfloat16),
            jax.ShapeDtypeStruct((S, HP), jnp.bfloat16),
            jax.ShapeDtypeStruct((S, E * H), jnp.float32),
        ],
        compiler_params=pltpu.CompilerParams(
            dimension_semantics=("parallel",)),
    )(rs, xbf, wqb, wkb, svt, sot, vmat, e12)

    parts = []
    for ci in range(4):
        q_start, klen = 2 * ci, (2 * ci + 2) * TS
        parts.append(pl.pallas_call(
            _make_attn_body(q_start, klen),
            grid=(H // 4, 2),
            in_specs=[
                pl.BlockSpec((TS, 4 * P),
                             lambda h, i, qs=q_start: (qs + i, h)),
                pl.BlockSpec((klen, 4 * P), lambda h, i: (0, h)),
                pl.BlockSpec((klen, 4 * P), lambda h, i: (0, h)),
            ],
            out_specs=pl.BlockSpec((TS, 4 * P), lambda h, i: (i, h)),
            out_shape=jax.ShapeDtypeStruct((2 * TS, HP), jnp.bfloat16),
            compiler_params=pltpu.CompilerParams(
                dimension_semantics=("parallel", "parallel")),
        )(qk, kk, vmixk))
    res = jnp.concatenate(parts, axis=0)

    out = pl.pallas_call(
        _oproj_body,
        grid=(NT,),
        in_specs=[
            pl.BlockSpec((TS, HP), lambda i: (i, 0)),
            pl.BlockSpec((TS, E * H), lambda i: (i, 0)),
            full((E, HP, D)),
            full((H, HP)),
        ],
        out_specs=pl.BlockSpec((TS, D), lambda i: (i, 0)),
        out_shape=jax.ShapeDtypeStruct((S, D), jnp.float32),
        compiler_params=pltpu.CompilerParams(
            dimension_semantics=("parallel",)),
    )(res, gok, omat, e12)

    return out.reshape(B, S, D)


# speedup vs baseline: 1.0277x; 1.0160x over previous
import math

import jax
import jax.numpy as jnp
from jax.experimental import pallas as pl
from jax.experimental.pallas import tpu as pltpu

B, S, D = 1, 2048, 768
H, E, TOPK, P = 12, 8, 2, 64
TS = 256
NT = S // TS
TSP = 512
NTP = S // TSP
HP = H * P

_SCALE = 1.0 / math.sqrt(P)
_S = math.sqrt(_SCALE)

_HI = jax.lax.Precision.HIGHEST


def _prep_body(v_ref, o_ref, x_ref, vmat_ref, omat_ref, xb_ref):
    for h in range(H):
        vmat_ref[0, :, h * P:(h + 1) * P] = v_ref[h, 0].astype(jnp.bfloat16)
        omat_ref[0, h * P:(h + 1) * P, :] = o_ref[h, 0].astype(jnp.bfloat16)
    xb_ref[...] = x_ref[...].astype(jnp.bfloat16)


def _top2_gates(logits, rs_over):
    probs = [jax.nn.sigmoid(logits[:, e * H:(e + 1) * H]) for e in range(E)]
    m1 = probs[0]
    for e in range(1, E):
        m1 = jnp.maximum(m1, probs[e])
    i1 = jnp.full(probs[0].shape, E, dtype=jnp.int32)
    for e in range(E - 1, -1, -1):
        i1 = jnp.where(probs[e] == m1, e, i1)
    neg = jnp.float32(-jnp.inf)
    q = [jnp.where(i1 == e, neg, probs[e]) for e in range(E)]
    m2 = q[0]
    for e in range(1, E):
        m2 = jnp.maximum(m2, q[e])
    i2 = jnp.full(probs[0].shape, E, dtype=jnp.int32)
    for e in range(E - 1, -1, -1):
        i2 = jnp.where(q[e] == m2, e, i2)
    denom = jnp.maximum(m1 + m2, jnp.float32(1e-9))
    scale = rs_over / denom
    gates = []
    for e in range(E):
        sel = jnp.logical_or(i1 == e, i2 == e)
        gates.append(jnp.where(sel, probs[e] * scale, jnp.float32(0.0)))
    return gates


def _proj_route_body(rs_ref, x_ref, wq_ref, wk_ref, svt_ref, sot_ref,
                     vmat_ref, e12_ref,
                     q_ref, k_ref, vmix_ref, go_ref):
    xb = x_ref[...]
    s = jnp.float32(_S)
    dn = (((1,), (0,)), ((), ()))
    dnt = (((1,), (1,)), ((), ()))
    q = jax.lax.dot_general(xb, wq_ref[...], dnt,
                            preferred_element_type=jnp.float32)
    q_ref[...] = (q * s).astype(jnp.bfloat16)
    k = jax.lax.dot_general(xb, wk_ref[...], dnt,
                            preferred_element_type=jnp.float32)
    k_ref[...] = (k * s).astype(jnp.bfloat16)

    rs = rs_ref[0, 0]
    lv = jax.lax.dot_general(xb, svt_ref[...], dnt,
                             preferred_element_type=jnp.float32)
    gv = _top2_gates(lv, rs)
    lo = jax.lax.dot_general(xb, sot_ref[...], dnt,
                             preferred_element_type=jnp.float32)
    go = _top2_gates(lo, rs)
    for e in range(E):
        go_ref[:, e * H:(e + 1) * H] = go[e]

    e12 = e12_ref[...]
    acc = jnp.zeros((TSP, HP), jnp.float32)
    for e in range(E):
        av = jax.lax.dot_general(xb, vmat_ref[e], dn,
                                 preferred_element_type=jnp.float32)
        gexp = _expand_gate(gv[e], e12)
        acc = acc + av * gexp
    vmix_ref[...] = acc.astype(jnp.bfloat16)


def _expand_gate(g, e12):
    return jax.lax.dot_general(g.astype(jnp.bfloat16), e12,
                               (((1,), (0,)), ((), ())),
                               preferred_element_type=jnp.float32)


def _make_attn_body(q_start, klen):
    def body(q_ref, k_ref, v_ref, o_ref):
        i = pl.program_id(1)
        row = (q_start + i) * TS + jax.lax.broadcasted_iota(
            jnp.int32, (TS, klen), 0)
        col = jax.lax.broadcasted_iota(jnp.int32, (TS, klen), 1)
        mask = col <= row
        for j in range(4):
            qv = q_ref[:, j * P:(j + 1) * P]
            kv = k_ref[:, j * P:(j + 1) * P]
            s = jax.lax.dot_general(qv, kv, (((1,), (1,)), ((), ())),
                                    preferred_element_type=jnp.float32)
            s = jnp.where(mask, s, jnp.float32(-1e30))
            m = jnp.max(s, axis=1, keepdims=True)
            p = jnp.exp(s - m)
            l = jnp.sum(p, axis=1, keepdims=True)
            p = (p / l).astype(jnp.bfloat16)
            o = jax.lax.dot_general(p, v_ref[:, j * P:(j + 1) * P],
                                    (((1,), (0,)), ((), ())),
                                    preferred_element_type=jnp.float32)
            o_ref[:, j * P:(j + 1) * P] = o.astype(jnp.bfloat16)
    return body


def _oproj_body(res_ref, go_ref, omat_ref, e12_ref, out_ref):
    res = res_ref[...].astype(jnp.float32)
    e12 = e12_ref[...]
    dn = (((1,), (0,)), ((), ()))
    acc = jnp.zeros((TSP, D), jnp.float32)
    for e in range(E):
        gexp = _expand_gate(go_ref[:, e * H:(e + 1) * H], e12)
        wres = (res * gexp).astype(jnp.bfloat16)
        acc = acc + jax.lax.dot_general(wres, omat_ref[e], dn,
                                        preferred_element_type=jnp.float32)
    out_ref[...] = acc


@jax.jit
def kernel(x, Wq, Wk, v, o, sel_v, sel_o, route_scale):
    wqb = Wq.astype(jnp.bfloat16)
    wkb = Wk.astype(jnp.bfloat16)
    svt = sel_v.reshape(H, E, D).transpose(1, 0, 2).reshape(E * H, D)
    svt = svt.astype(jnp.bfloat16)
    sot = sel_o.reshape(H, E, D).transpose(1, 0, 2).reshape(E * H, D)
    sot = sot.astype(jnp.bfloat16)
    rs = route_scale.reshape(1, 1)
    e12 = jnp.repeat(jnp.eye(H, dtype=jnp.bfloat16), P, axis=1)

    def full(shape):
        return pl.BlockSpec(shape, lambda *_: (0,) * len(shape))

    vmat, omat, xbf = pl.pallas_call(
        _prep_body,
        grid=(E,),
        in_specs=[
            pl.BlockSpec((H, 1, D, P), lambda e: (0, e, 0, 0)),
            pl.BlockSpec((H, 1, P, D), lambda e: (0, e, 0, 0)),
            pl.BlockSpec((TS, D), lambda e: (e, 0)),
        ],
        out_specs=[
            pl.BlockSpec((1, D, HP), lambda e: (e, 0, 0)),
            pl.BlockSpec((1, HP, D), lambda e: (e, 0, 0)),
            pl.BlockSpec((TS, D), lambda e: (e, 0)),
        ],
        out_shape=[
            jax.ShapeDtypeStruct((E, D, HP), jnp.bfloat16),
            jax.ShapeDtypeStruct((E, HP, D), jnp.bfloat16),
            jax.ShapeDtypeStruct((S, D), jnp.bfloat16),
        ],
        compiler_params=pltpu.CompilerParams(
            dimension_semantics=("parallel",)),
    )(v.reshape(H, E, D, P), o.reshape(H, E, P, D), x[0])

    qk, kk, vmixk, gok = pl.pallas_call(
        _proj_route_body,
        grid=(NTP,),
        in_specs=[
            pl.BlockSpec(memory_space=pltpu.SMEM),
            pl.BlockSpec((TSP, D), lambda i: (i, 0)),
            full((HP, D)),
            full((HP, D)),
            full((E * H, D)),
            full((E * H, D)),
            full((E, D, HP)),
            full((H, HP)),
        ],
        out_specs=[
            pl.BlockSpec((TSP, HP), lambda i: (i, 0)),
            pl.BlockSpec((TSP, HP), lambda i: (i, 0)),
            pl.BlockSpec((TSP, HP), lambda i: (i, 0)),
            pl.BlockSpec((TSP, E * H), lambda i: (i, 0)),
        ],
        out_shape=[
            jax.ShapeDtypeStruct((S, HP), jnp.bfloat16),
            jax.ShapeDtypeStruct((S, HP), jnp.bfloat16),
            jax.ShapeDtypeStruct((S, HP), jnp.bfloat16),
            jax.ShapeDtypeStruct((S, E * H), jnp.float32),
        ],
        compiler_params=pltpu.CompilerParams(
            dimension_semantics=("parallel",)),
    )(rs, xbf, wqb, wkb, svt, sot, vmat, e12)

    parts = []
    for ci in range(4):
        q_start, klen = 2 * ci, (2 * ci + 2) * TS
        parts.append(pl.pallas_call(
            _make_attn_body(q_start, klen),
            grid=(H // 4, 2),
            in_specs=[
                pl.BlockSpec((TS, 4 * P),
                             lambda h, i, qs=q_start: (qs + i, h)),
                pl.BlockSpec((klen, 4 * P), lambda h, i: (0, h)),
                pl.BlockSpec((klen, 4 * P), lambda h, i: (0, h)),
            ],
            out_specs=pl.BlockSpec((TS, 4 * P), lambda h, i: (i, h)),
            out_shape=jax.ShapeDtypeStruct((2 * TS, HP), jnp.bfloat16),
            compiler_params=pltpu.CompilerParams(
                dimension_semantics=("parallel", "parallel")),
        )(qk, kk, vmixk))
    res = jnp.concatenate(parts, axis=0)

    out = pl.pallas_call(
        _oproj_body,
        grid=(NTP,),
        in_specs=[
            pl.BlockSpec((TSP, HP), lambda i: (i, 0)),
            pl.BlockSpec((TSP, E * H), lambda i: (i, 0)),
            full((E, HP, D)),
            full((H, HP)),
        ],
        out_specs=pl.BlockSpec((TSP, D), lambda i: (i, 0)),
        out_shape=jax.ShapeDtypeStruct((S, D), jnp.float32),
        compiler_params=pltpu.CompilerParams(
            dimension_semantics=("parallel",)),
    )(res, gok, omat, e12)

    return out.reshape(B, S, D)


# speedup vs baseline: 1.0366x; 1.0087x over previous
import math

import jax
import jax.numpy as jnp
from jax.experimental import pallas as pl
from jax.experimental.pallas import tpu as pltpu

B, S, D = 1, 2048, 768
H, E, TOPK, P = 12, 8, 2, 64
TS = 256
NT = S // TS
TSP = 512
NTP = S // TSP
HP = H * P

_SCALE = 1.0 / math.sqrt(P)
_S = math.sqrt(_SCALE)

_HI = jax.lax.Precision.HIGHEST


def _prep_body(v_ref, o_ref, x_ref, vmat_ref, omat_ref, xb_ref):
    for h in range(H):
        vmat_ref[0, :, h * P:(h + 1) * P] = v_ref[h, 0].astype(jnp.bfloat16)
        omat_ref[0, h * P:(h + 1) * P, :] = o_ref[h, 0].astype(jnp.bfloat16)
    xb_ref[...] = x_ref[...].astype(jnp.bfloat16)


def _top2_gates(logits, rs_over):
    probs = [jax.nn.sigmoid(logits[:, e * H:(e + 1) * H]) for e in range(E)]
    m1 = probs[0]
    for e in range(1, E):
        m1 = jnp.maximum(m1, probs[e])
    i1 = jnp.full(probs[0].shape, E, dtype=jnp.int32)
    for e in range(E - 1, -1, -1):
        i1 = jnp.where(probs[e] == m1, e, i1)
    neg = jnp.float32(-jnp.inf)
    q = [jnp.where(i1 == e, neg, probs[e]) for e in range(E)]
    m2 = q[0]
    for e in range(1, E):
        m2 = jnp.maximum(m2, q[e])
    i2 = jnp.full(probs[0].shape, E, dtype=jnp.int32)
    for e in range(E - 1, -1, -1):
        i2 = jnp.where(q[e] == m2, e, i2)
    denom = jnp.maximum(m1 + m2, jnp.float32(1e-9))
    scale = rs_over / denom
    gates = []
    for e in range(E):
        sel = jnp.logical_or(i1 == e, i2 == e)
        gates.append(jnp.where(sel, probs[e] * scale, jnp.float32(0.0)))
    return gates


def _proj_route_body(rs_ref, x_ref, wq_ref, wk_ref, svt_ref, sot_ref,
                     vmat_ref, e12_ref,
                     q_ref, k_ref, vmix_ref, go_ref):
    xb = x_ref[...]
    s = jnp.float32(_S)
    dn = (((1,), (0,)), ((), ()))
    dnt = (((1,), (1,)), ((), ()))
    q = jax.lax.dot_general(xb, wq_ref[...], dnt,
                            preferred_element_type=jnp.float32)
    q_ref[...] = (q * s).astype(jnp.bfloat16)
    k = jax.lax.dot_general(xb, wk_ref[...], dnt,
                            preferred_element_type=jnp.float32)
    k_ref[...] = (k * s).astype(jnp.bfloat16)

    rs = rs_ref[0, 0]
    lv = jax.lax.dot_general(xb, svt_ref[...], dnt,
                             preferred_element_type=jnp.float32)
    gv = _top2_gates(lv, rs)
    lo = jax.lax.dot_general(xb, sot_ref[...], dnt,
                             preferred_element_type=jnp.float32)
    go = _top2_gates(lo, rs)
    for e in range(E):
        go_ref[:, e * H:(e + 1) * H] = go[e]

    e12 = e12_ref[...]
    acc = jnp.zeros((TSP, HP), jnp.float32)
    for e in range(E):
        av = jax.lax.dot_general(xb, vmat_ref[e], dn,
                                 preferred_element_type=jnp.float32)
        gexp = _expand_gate(gv[e], e12)
        acc = acc + av * gexp
    vmix_ref[...] = acc.astype(jnp.bfloat16)


def _expand_gate(g, e12):
    return jax.lax.dot_general(g.astype(jnp.bfloat16), e12,
                               (((1,), (0,)), ((), ())),
                               preferred_element_type=jnp.float32)


def _make_attn_body(q_start, klen):
    def body(q_ref, k_ref, v_ref, o_ref):
        i = pl.program_id(1)
        row = (q_start + i) * TS + jax.lax.broadcasted_iota(
            jnp.int32, (TS, klen), 0)
        col = jax.lax.broadcasted_iota(jnp.int32, (TS, klen), 1)
        mask = col <= row
        for j in range(4):
            qv = q_ref[:, j * P:(j + 1) * P]
            kv = k_ref[:, j * P:(j + 1) * P]
            s = jax.lax.dot_general(qv, kv, (((1,), (1,)), ((), ())),
                                    preferred_element_type=jnp.float32)
            s = jnp.where(mask, s, jnp.float32(-1e30))
            m = jnp.max(s, axis=1, keepdims=True)
            p = jnp.exp(s - m)
            l = jnp.sum(p, axis=1, keepdims=True)
            p = (p / l).astype(jnp.bfloat16)
            o = jax.lax.dot_general(p, v_ref[:, j * P:(j + 1) * P],
                                    (((1,), (0,)), ((), ())),
                                    preferred_element_type=jnp.float32)
            o_ref[:, j * P:(j + 1) * P] = o.astype(jnp.bfloat16)
    return body


def _oproj_body(res_ref, go_ref, omat_ref, e12_ref, out_ref):
    res = res_ref[...].astype(jnp.float32)
    e12 = e12_ref[...]
    dn = (((1,), (0,)), ((), ()))
    acc = jnp.zeros((TSP, D), jnp.float32)
    for e in range(E):
        gexp = _expand_gate(go_ref[:, e * H:(e + 1) * H], e12)
        wres = (res * gexp).astype(jnp.bfloat16)
        acc = acc + jax.lax.dot_general(wres, omat_ref[e], dn,
                                        preferred_element_type=jnp.float32)
    out_ref[...] = acc


@jax.jit
def kernel(x, Wq, Wk, v, o, sel_v, sel_o, route_scale):
    wqb = Wq.astype(jnp.bfloat16)
    wkb = Wk.astype(jnp.bfloat16)
    svt = sel_v.reshape(H, E, D).transpose(1, 0, 2).reshape(E * H, D)
    svt = svt.astype(jnp.bfloat16)
    sot = sel_o.reshape(H, E, D).transpose(1, 0, 2).reshape(E * H, D)
    sot = sot.astype(jnp.bfloat16)
    rs = route_scale.reshape(1, 1)
    e12 = jnp.repeat(jnp.eye(H, dtype=jnp.bfloat16), P, axis=1)

    def full(shape):
        return pl.BlockSpec(shape, lambda *_: (0,) * len(shape))

    vmat, omat, xbf = pl.pallas_call(
        _prep_body,
        grid=(E,),
        in_specs=[
            pl.BlockSpec((H, 1, D, P), lambda e: (0, e, 0, 0)),
            pl.BlockSpec((H, 1, P, D), lambda e: (0, e, 0, 0)),
            pl.BlockSpec((TS, D), lambda e: (e, 0)),
        ],
        out_specs=[
            pl.BlockSpec((1, D, HP), lambda e: (e, 0, 0)),
            pl.BlockSpec((1, HP, D), lambda e: (e, 0, 0)),
            pl.BlockSpec((TS, D), lambda e: (e, 0)),
        ],
        out_shape=[
            jax.ShapeDtypeStruct((E, D, HP), jnp.bfloat16),
            jax.ShapeDtypeStruct((E, HP, D), jnp.bfloat16),
            jax.ShapeDtypeStruct((S, D), jnp.bfloat16),
        ],
        compiler_params=pltpu.CompilerParams(
            dimension_semantics=("parallel",)),
    )(v.reshape(H, E, D, P), o.reshape(H, E, P, D), x[0])

    qk, kk, vmixk, gok = pl.pallas_call(
        _proj_route_body,
        grid=(NTP,),
        in_specs=[
            pl.BlockSpec(memory_space=pltpu.SMEM),
            pl.BlockSpec((TSP, D), lambda i: (i, 0)),
            full((HP, D)),
            full((HP, D)),
            full((E * H, D)),
            full((E * H, D)),
            full((E, D, HP)),
            full((H, HP)),
        ],
        out_specs=[
            pl.BlockSpec((TSP, HP), lambda i: (i, 0)),
            pl.BlockSpec((TSP, HP), lambda i: (i, 0)),
            pl.BlockSpec((TSP, HP), lambda i: (i, 0)),
            pl.BlockSpec((TSP, E * H), lambda i: (i, 0)),
        ],
        out_shape=[
            jax.ShapeDtypeStruct((S, HP), jnp.bfloat16),
            jax.ShapeDtypeStruct((S, HP), jnp.bfloat16),
            jax.ShapeDtypeStruct((S, HP), jnp.bfloat16),
            jax.ShapeDtypeStruct((S, E * H), jnp.float32),
        ],
        compiler_params=pltpu.CompilerParams(
            dimension_semantics=("parallel",)),
    )(rs, xbf, wqb, wkb, svt, sot, vmat, e12)

    parts = []
    for ci in range(NT):
        q_start, klen = ci, (ci + 1) * TS
        parts.append(pl.pallas_call(
            _make_attn_body(q_start, klen),
            grid=(H // 4, 1),
            in_specs=[
                pl.BlockSpec((TS, 4 * P),
                             lambda h, i, qs=q_start: (qs + i, h)),
                pl.BlockSpec((klen, 4 * P), lambda h, i: (0, h)),
                pl.BlockSpec((klen, 4 * P), lambda h, i: (0, h)),
            ],
            out_specs=pl.BlockSpec((TS, 4 * P), lambda h, i: (i, h)),
            out_shape=jax.ShapeDtypeStruct((TS, HP), jnp.bfloat16),
            compiler_params=pltpu.CompilerParams(
                dimension_semantics=("parallel", "parallel")),
        )(qk, kk, vmixk))
    res = jnp.concatenate(parts, axis=0)

    out = pl.pallas_call(
        _oproj_body,
        grid=(NTP,),
        in_specs=[
            pl.BlockSpec((TSP, HP), lambda i: (i, 0)),
            pl.BlockSpec((TSP, E * H), lambda i: (i, 0)),
            full((E, HP, D)),
            full((H, HP)),
        ],
        out_specs=pl.BlockSpec((TSP, D), lambda i: (i, 0)),
        out_shape=jax.ShapeDtypeStruct((S, D), jnp.float32),
        compiler_params=pltpu.CompilerParams(
            dimension_semantics=("parallel",)),
    )(res, gok, omat, e12)

    return out.reshape(B, S, D)


# speedup vs baseline: 1.0370x; 1.0004x over previous
import math

import jax
import jax.numpy as jnp
from jax.experimental import pallas as pl
from jax.experimental.pallas import tpu as pltpu

B, S, D = 1, 2048, 768
H, E, P = 12, 8, 64
TS = 256
NT = S // TS
TSP = 512
NTP = S // TSP
HP = H * P

_SCALE = 1.0 / math.sqrt(P)
_S = math.sqrt(_SCALE)


def _prep_body(v_ref, o_ref, x_ref, vmat_ref, omat_ref, xb_ref):
    for h in range(H):
        vmat_ref[0, :, h * P:(h + 1) * P] = v_ref[h, 0].astype(jnp.bfloat16)
        omat_ref[0, h * P:(h + 1) * P, :] = o_ref[h, 0].astype(jnp.bfloat16)
    xb_ref[...] = x_ref[...].astype(jnp.bfloat16)


def _top2_gates(logits, rs_over):
    probs = [jax.nn.sigmoid(logits[:, e * H:(e + 1) * H]) for e in range(E)]
    m1 = probs[0]
    for e in range(1, E):
        m1 = jnp.maximum(m1, probs[e])
    i1 = jnp.full(probs[0].shape, E, dtype=jnp.int32)
    for e in range(E - 1, -1, -1):
        i1 = jnp.where(probs[e] == m1, e, i1)
    neg = jnp.float32(-jnp.inf)
    q = [jnp.where(i1 == e, neg, probs[e]) for e in range(E)]
    m2 = q[0]
    for e in range(1, E):
        m2 = jnp.maximum(m2, q[e])
    i2 = jnp.full(probs[0].shape, E, dtype=jnp.int32)
    for e in range(E - 1, -1, -1):
        i2 = jnp.where(q[e] == m2, e, i2)
    denom = jnp.maximum(m1 + m2, jnp.float32(1e-9))
    scale = rs_over / denom
    gates = []
    for e in range(E):
        sel = jnp.logical_or(i1 == e, i2 == e)
        gates.append(jnp.where(sel, probs[e] * scale, jnp.float32(0.0)))
    return gates


def _proj_route_body(rs_ref, x_ref, wq_ref, wk_ref, svt_ref, sot_ref,
                     vmat_ref, e12_ref,
                     q_ref, k_ref, vmix_ref, go_ref):
    xb = x_ref[...]
    s = jnp.float32(_S)
    dn = (((1,), (0,)), ((), ()))
    dnt = (((1,), (1,)), ((), ()))
    q = jax.lax.dot_general(xb, wq_ref[...], dnt,
                            preferred_element_type=jnp.float32)
    q_ref[...] = (q * s).astype(jnp.bfloat16)
    k = jax.lax.dot_general(xb, wk_ref[...], dnt,
                            preferred_element_type=jnp.float32)
    k_ref[...] = (k * s).astype(jnp.bfloat16)

    rs = rs_ref[0, 0]
    lv = jax.lax.dot_general(xb, svt_ref[...], dnt,
                             preferred_element_type=jnp.float32)
    gv = _top2_gates(lv, rs)
    lo = jax.lax.dot_general(xb, sot_ref[...], dnt,
                             preferred_element_type=jnp.float32)
    go = _top2_gates(lo, rs)
    for e in range(E):
        go_ref[:, e * H:(e + 1) * H] = go[e]

    e12 = e12_ref[...]
    acc = jnp.zeros((TSP, HP), jnp.float32)
    for e in range(E):
        av = jax.lax.dot_general(xb, vmat_ref[e], dn,
                                 preferred_element_type=jnp.float32)
        gexp = _expand_gate(gv[e], e12)
        acc = acc + av * gexp
    vmix_ref[...] = acc.astype(jnp.bfloat16)


def _expand_gate(g, e12):
    return jax.lax.dot_general(g.astype(jnp.bfloat16), e12,
                               (((1,), (0,)), ((), ())),
                               preferred_element_type=jnp.float32)


def _make_attn_body(q_start, klen):
    def body(q_ref, k_ref, v_ref, o_ref):
        i = pl.program_id(1)
        row = (q_start + i) * TS + jax.lax.broadcasted_iota(
            jnp.int32, (TS, klen), 0)
        col = jax.lax.broadcasted_iota(jnp.int32, (TS, klen), 1)
        mask = col <= row
        for j in range(4):
            qv = q_ref[:, j * P:(j + 1) * P]
            kv = k_ref[:, j * P:(j + 1) * P]
            s = jax.lax.dot_general(qv, kv, (((1,), (1,)), ((), ())),
                                    preferred_element_type=jnp.float32)
            s = jnp.where(mask, s, jnp.float32(-1e30))
            m = jnp.max(s, axis=1, keepdims=True)
            p = jnp.exp(s - m)
            l = jnp.sum(p, axis=1, keepdims=True)
            p = (p / l).astype(jnp.bfloat16)
            o = jax.lax.dot_general(p, v_ref[:, j * P:(j + 1) * P],
                                    (((1,), (0,)), ((), ())),
                                    preferred_element_type=jnp.float32)
            o_ref[:, j * P:(j + 1) * P] = o.astype(jnp.bfloat16)
    return body


def _oproj_body(res_ref, go_ref, omat_ref, e12_ref, out_ref):
    res = res_ref[...].astype(jnp.float32)
    e12 = e12_ref[...]
    dn = (((1,), (0,)), ((), ()))
    acc = jnp.zeros((TSP, D), jnp.float32)
    for e in range(E):
        gexp = _expand_gate(go_ref[:, e * H:(e + 1) * H], e12)
        wres = (res * gexp).astype(jnp.bfloat16)
        acc = acc + jax.lax.dot_general(wres, omat_ref[e], dn,
                                        preferred_element_type=jnp.float32)
    out_ref[...] = acc


@jax.jit
def kernel(x, Wq, Wk, v, o, sel_v, sel_o, route_scale):
    wqb = Wq.astype(jnp.bfloat16)
    wkb = Wk.astype(jnp.bfloat16)
    svt = sel_v.reshape(H, E, D).transpose(1, 0, 2).reshape(E * H, D)
    svt = svt.astype(jnp.bfloat16)
    sot = sel_o.reshape(H, E, D).transpose(1, 0, 2).reshape(E * H, D)
    sot = sot.astype(jnp.bfloat16)
    rs = route_scale.reshape(1, 1)
    e12 = jnp.repeat(jnp.eye(H, dtype=jnp.bfloat16), P, axis=1)

    def full(shape):
        return pl.BlockSpec(shape, lambda *_: (0,) * len(shape))

    vmat, omat, xbf = pl.pallas_call(
        _prep_body,
        grid=(E,),
        in_specs=[
            pl.BlockSpec((H, 1, D, P), lambda e: (0, e, 0, 0)),
            pl.BlockSpec((H, 1, P, D), lambda e: (0, e, 0, 0)),
            pl.BlockSpec((TS, D), lambda e: (e, 0)),
        ],
        out_specs=[
            pl.BlockSpec((1, D, HP), lambda e: (e, 0, 0)),
            pl.BlockSpec((1, HP, D), lambda e: (e, 0, 0)),
            pl.BlockSpec((TS, D), lambda e: (e, 0)),
        ],
        out_shape=[
            jax.ShapeDtypeStruct((E, D, HP), jnp.bfloat16),
            jax.ShapeDtypeStruct((E, HP, D), jnp.bfloat16),
            jax.ShapeDtypeStruct((S, D), jnp.bfloat16),
        ],
        compiler_params=pltpu.CompilerParams(
            dimension_semantics=("parallel",)),
    )(v.reshape(H, E, D, P), o.reshape(H, E, P, D), x[0])

    qk, kk, vmixk, gok = pl.pallas_call(
        _proj_route_body,
        grid=(NTP,),
        in_specs=[
            pl.BlockSpec(memory_space=pltpu.SMEM),
            pl.BlockSpec((TSP, D), lambda i: (i, 0)),
            full((HP, D)),
            full((HP, D)),
            full((E * H, D)),
            full((E * H, D)),
            full((E, D, HP)),
            full((H, HP)),
        ],
        out_specs=[
            pl.BlockSpec((TSP, HP), lambda i: (i, 0)),
            pl.BlockSpec((TSP, HP), lambda i: (i, 0)),
            pl.BlockSpec((TSP, HP), lambda i: (i, 0)),
            pl.BlockSpec((TSP, E * H), lambda i: (i, 0)),
        ],
        out_shape=[
            jax.ShapeDtypeStruct((S, HP), jnp.bfloat16),
            jax.ShapeDtypeStruct((S, HP), jnp.bfloat16),
            jax.ShapeDtypeStruct((S, HP), jnp.bfloat16),
            jax.ShapeDtypeStruct((S, E * H), jnp.float32),
        ],
        compiler_params=pltpu.CompilerParams(
            dimension_semantics=("parallel",)),
    )(rs, xbf, wqb, wkb, svt, sot, vmat, e12)

    parts = []
    for ci in range(NT):
        q_start, klen = ci, (ci + 1) * TS
        parts.append(pl.pallas_call(
            _make_attn_body(q_start, klen),
            grid=(H // 4, 1),
            in_specs=[
                pl.BlockSpec((TS, 4 * P),
                             lambda h, i, qs=q_start: (qs + i, h)),
                pl.BlockSpec((klen, 4 * P), lambda h, i: (0, h)),
                pl.BlockSpec((klen, 4 * P), lambda h, i: (0, h)),
            ],
            out_specs=pl.BlockSpec((TS, 4 * P), lambda h, i: (i, h)),
            out_shape=jax.ShapeDtypeStruct((TS, HP), jnp.bfloat16),
            compiler_params=pltpu.CompilerParams(
                dimension_semantics=("parallel", "parallel")),
        )(qk, kk, vmixk))
    res = jnp.concatenate(parts, axis=0)

    out = pl.pallas_call(
        _oproj_body,
        grid=(NTP,),
        in_specs=[
            pl.BlockSpec((TSP, HP), lambda i: (i, 0)),
            pl.BlockSpec((TSP, E * H), lambda i: (i, 0)),
            full((E, HP, D)),
            full((H, HP)),
        ],
        out_specs=pl.BlockSpec((TSP, D), lambda i: (i, 0)),
        out_shape=jax.ShapeDtypeStruct((S, D), jnp.float32),
        compiler_params=pltpu.CompilerParams(
            dimension_semantics=("parallel",)),
    )(res, gok, omat, e12)

    return out.reshape(B, S, D)
